# baseline (speedup 1.0000x reference)
import functools

import jax
import jax.numpy as jnp
from jax import lax
from jax.experimental import pallas as pl
from jax.experimental.pallas import tpu as pltpu
from jax.experimental.pallas import tpu_sc as plsc

_D = 128
_NW = 32
_GE = 40
_GB = 64
_SC_PARAMS = pltpu.CompilerParams(needs_layout_passes=False)


def _mm_body(x_ref, w_ref, b_ref, o_ref):
    o_ref[...] = (
        jnp.dot(x_ref[...], w_ref[...], preferred_element_type=jnp.float32)
        + b_ref[...]
    )


def _mm(x, w, b, br=1000):
    r, d = x.shape
    k = w.shape[1]
    assert r % br == 0
    return pl.pallas_call(
        _mm_body,
        grid=(r // br,),
        in_specs=[
            pl.BlockSpec((br, d), lambda i: (i, 0)),
            pl.BlockSpec((d, k), lambda i: (0, 0)),
            pl.BlockSpec((1, k), lambda i: (0, 0)),
        ],
        out_specs=pl.BlockSpec((br, k), lambda i: (i, 0)),
        out_shape=jax.ShapeDtypeStruct((r, k), jnp.float32),
    )(x, w, b.reshape(1, k))


def _tables_body(x_ref, wgu_ref, bgu_ref, wgd_ref, bgd_ref, whd_ref, bhd_ref,
                 gu_ref, gd_ref, hd_ref):
    x = x_ref[...]
    gu_ref[...] = jnp.dot(x, wgu_ref[...], preferred_element_type=jnp.float32) + bgu_ref[...]
    gd_ref[...] = jnp.dot(x, wgd_ref[...], preferred_element_type=jnp.float32) + bgd_ref[...]
    hd_ref[...] = jnp.dot(x, whd_ref[...], preferred_element_type=jnp.float32) + bhd_ref[...]


def _node_tables(x, p, br=1000):
    r, d = x.shape
    assert r % br == 0
    wgu = jnp.concatenate([p['Wsg'], p['Wsu']], axis=1)
    bgu = jnp.concatenate([p['bsg'], p['bsu']]).reshape(1, 2 * d)
    return pl.pallas_call(
        _tables_body,
        grid=(r // br,),
        in_specs=[
            pl.BlockSpec((br, d), lambda i: (i, 0)),
            pl.BlockSpec((d, 2 * d), lambda i: (0, 0)),
            pl.BlockSpec((1, 2 * d), lambda i: (0, 0)),
            pl.BlockSpec((d, d), lambda i: (0, 0)),
            pl.BlockSpec((1, d), lambda i: (0, 0)),
            pl.BlockSpec((d, d), lambda i: (0, 0)),
            pl.BlockSpec((1, d), lambda i: (0, 0)),
        ],
        out_specs=[
            pl.BlockSpec((br, 2 * d), lambda i: (i, 0)),
            pl.BlockSpec((br, d), lambda i: (i, 0)),
            pl.BlockSpec((br, d), lambda i: (i, 0)),
        ],
        out_shape=[
            jax.ShapeDtypeStruct((r, 2 * d), jnp.float32),
            jax.ShapeDtypeStruct((r, d), jnp.float32),
            jax.ShapeDtypeStruct((r, d), jnp.float32),
        ],
    )(x, wgu, bgu, p['Wdg'], p['bdg'].reshape(1, d), p['Wdu'], p['bdu'].reshape(1, d))


def _ln(x, g=None, b=None, eps=1e-5):
    m = jnp.mean(x, axis=-1, keepdims=True)
    v = jnp.mean((x - m) ** 2, axis=-1, keepdims=True)
    y = (x - m) * lax.rsqrt(v + eps)
    if g is not None:
        y = y * g + b
    return y


def _silu(x):
    return x * jax.nn.sigmoid(x)


def _node_upd_t2_body(hd_ref, ad_ref, node_ref, g_ref, b_ref, w_ref, be_ref,
                      bl_ref, t2_ref):
    d = hd_ref.shape[-1]
    ad = ad_ref[...]
    h = hd_ref[...] + ad[:, :d] / (ad[:, d:] + 1e-6)
    bl = node_ref[...] + _silu(_ln(h, g_ref[...], b_ref[...]))
    bl_ref[...] = bl
    t2_ref[...] = jnp.dot(bl, w_ref[...], preferred_element_type=jnp.float32) + be_ref[...]


def _node_upd_t2(hd, ad, node, g, b, weg, beg, br=1000):
    r, d = hd.shape
    return pl.pallas_call(
        _node_upd_t2_body,
        grid=(r // br,),
        in_specs=[
            pl.BlockSpec((br, d), lambda i: (i, 0)),
            pl.BlockSpec((br, 2 * d), lambda i: (i, 0)),
            pl.BlockSpec((br, d), lambda i: (i, 0)),
            pl.BlockSpec((1, d), lambda i: (0, 0)),
            pl.BlockSpec((1, d), lambda i: (0, 0)),
            pl.BlockSpec((d, d), lambda i: (0, 0)),
            pl.BlockSpec((1, d), lambda i: (0, 0)),
        ],
        out_specs=[
            pl.BlockSpec((br, d), lambda i: (i, 0)),
            pl.BlockSpec((br, d), lambda i: (i, 0)),
        ],
        out_shape=[
            jax.ShapeDtypeStruct((r, d), jnp.float32),
            jax.ShapeDtypeStruct((r, d), jnp.float32),
        ],
    )(hd, ad, node, g.reshape(1, d), b.reshape(1, d), weg, beg.reshape(1, d))


def _fusion_body(atom_ref, hd_ref, ad2_ref, ad3_ref, g_ref, b_ref,
                 wf1_ref, bf1_ref, g1_ref, b1_ref, wf2_ref, bf2_ref, o_ref):
    d = hd_ref.shape[-1]
    atom = atom_ref[...]
    hd = hd_ref[...]
    g, b = g_ref[...], b_ref[...]

    ad2 = ad2_ref[...]
    h2 = hd + ad2[:, :d] / (ad2[:, d:] + 1e-6)
    g2l = _ln(atom + _silu(_ln(h2, g, b)))

    ad3 = ad3_ref[...]
    h3 = hd + ad3[:, :d] / (ad3[:, d:] + 1e-6)
    glob = _ln(atom + _silu(_ln(h3, g, b)))

    gate = jnp.concatenate([g2l, glob], axis=-1)
    h = jnp.dot(gate, wf1_ref[...], preferred_element_type=jnp.float32) + bf1_ref[...]
    h = jax.nn.relu(_ln(h, g1_ref[...], b1_ref[...]))
    z = jax.nn.sigmoid(
        jnp.dot(h, wf2_ref[...], preferred_element_type=jnp.float32) + bf2_ref[...])
    o_ref[...] = z * g2l + (1.0 - z) * glob


def _fusion(atom, hd, ad2, ad3, g, b, f, br=1000):
    r, d = atom.shape
    return pl.pallas_call(
        _fusion_body,
        grid=(r // br,),
        in_specs=[
            pl.BlockSpec((br, d), lambda i: (i, 0)),
            pl.BlockSpec((br, d), lambda i: (i, 0)),
            pl.BlockSpec((br, 2 * d), lambda i: (i, 0)),
            pl.BlockSpec((br, 2 * d), lambda i: (i, 0)),
            pl.BlockSpec((1, d), lambda i: (0, 0)),
            pl.BlockSpec((1, d), lambda i: (0, 0)),
            pl.BlockSpec((2 * d, d), lambda i: (0, 0)),
            pl.BlockSpec((1, d), lambda i: (0, 0)),
            pl.BlockSpec((1, d), lambda i: (0, 0)),
            pl.BlockSpec((1, d), lambda i: (0, 0)),
            pl.BlockSpec((d, d), lambda i: (0, 0)),
            pl.BlockSpec((1, d), lambda i: (0, 0)),
        ],
        out_specs=pl.BlockSpec((br, d), lambda i: (i, 0)),
        out_shape=jax.ShapeDtypeStruct((r, d), jnp.float32),
    )(atom, hd, ad2, ad3, g.reshape(1, d), b.reshape(1, d),
      f['Wf1'], f['bf1'].reshape(1, d), f['g1'].reshape(1, d),
      f['b1'].reshape(1, d), f['Wf2'], f['bf2'].reshape(1, d))


def _rsqrt16(x):
    i = plsc.bitcast(x, jnp.int32)
    i = jnp.int32(0x5F3759DF) - (i >> 1)
    y = plsc.bitcast(i, jnp.float32)
    for _ in range(4):
        y = y * (1.5 - 0.5 * x * y * y)
    return y


def _sig16(x):
    return 1.0 / (1.0 + jnp.exp(-x))


def _edge_out_body(ef_ref, ep_ref, g_ref, b_ref, o_ref):
    o_ref[...] = ef_ref[...] + _silu(_ln(ep_ref[...], g_ref[...], b_ref[...]))


def _edge_out(ef, ep, g, b, br=1000):
    r, d = ef.shape
    return pl.pallas_call(
        _edge_out_body,
        grid=(r // br,),
        in_specs=[
            pl.BlockSpec((br, d), lambda i: (i, 0)),
            pl.BlockSpec((br, d), lambda i: (i, 0)),
            pl.BlockSpec((1, d), lambda i: (0, 0)),
            pl.BlockSpec((1, d), lambda i: (0, 0)),
        ],
        out_specs=pl.BlockSpec((br, d), lambda i: (i, 0)),
        out_shape=jax.ShapeDtypeStruct((r, d), jnp.float32),
    )(ef, ep, g.reshape(1, d), b.reshape(1, d))


def _edge_phase(gu, gd, t, src, dst, want_ep=True):
    bb = t.shape[0]
    bpw = bb // _NW
    nblk = bpw // _GE
    assert bpw % _GE == 0 and (bpw % 8) == 0
    mesh = plsc.VectorSubcoreMesh(core_axis_name="c", subcore_axis_name="s")

    def body(*refs):
        if want_ep:
            (gu_h, gd_h, t_h, src_h, dst_h, ms_h, ep_h,
             src_v, dst_v, gu_v, gd_v, t_v, ms_v, ep_v,
             sgu0, sgu1, sgd0, sgd1, st0, st1, swb0, swb1) = refs
        else:
            (gu_h, gd_h, t_h, src_h, dst_h, ms_h,
             src_v, dst_v, gu_v, gd_v, t_v, ms_v,
             sgu0, sgu1, sgd0, sgd1, st0, st1, swb0, swb1) = refs
        sems = [(sgu0, sgd0, st0), (sgu1, sgd1, st1)]
        swbs = [swb0, swb1]
        wid = lax.axis_index("s") * 2 + lax.axis_index("c")
        base = wid * bpw
        pltpu.sync_copy(src_h.at[pl.ds(base, bpw)], src_v)
        pltpu.sync_copy(dst_h.at[pl.ds(base, bpw)], dst_v)

        def fetch(j, buf):
            s1, s2, s3 = sems[buf]
            pltpu.async_copy(
                gu_h.at[src_v.at[pl.ds(j * _GE, _GE)]], gu_v.at[buf], s1)
            pltpu.async_copy(
                gd_h.at[dst_v.at[pl.ds(j * _GE, _GE)]], gd_v.at[buf], s2)
            pltpu.async_copy(
                t_h.at[pl.ds(base + j * _GE, _GE)], t_v.at[buf], s3)

        def wait_fetch(j, buf):
            s1, s2, s3 = sems[buf]
            pltpu.make_async_copy(
                gu_h.at[src_v.at[pl.ds(j * _GE, _GE)]], gu_v.at[buf],
                s1).wait()
            pltpu.make_async_copy(
                gd_h.at[dst_v.at[pl.ds(j * _GE, _GE)]], gd_v.at[buf],
                s2).wait()
            pltpu.make_async_copy(
                t_h.at[pl.ds(base + j * _GE, _GE)], t_v.at[buf], s3).wait()

        def compute(j, buf):
            @pl.when(j >= 2)
            def _():
                pltpu.make_async_copy(
                    ms_v.at[buf], ms_h.at[pl.ds(base, _GE)], swbs[buf]).wait()
                if want_ep:
                    pltpu.make_async_copy(
                        ep_v.at[buf], ep_h.at[pl.ds(base, _GE)],
                        swbs[buf]).wait()

            def row(r, _):
                ep, uu = [], []
                for c in range(8):
                    sl = pl.ds(c * 16, 16)
                    ep.append(gu_v[buf, r, sl] + gd_v[buf, r, sl]
                              + t_v[buf, r, sl])
                    uu.append(gu_v[buf, r, pl.ds(_D + c * 16, 16)])
                for c in range(8):
                    sl = pl.ds(c * 16, 16)
                    sig = _sig16(ep[c])
                    ms_v[buf, r, sl] = sig * uu[c]
                    ms_v[buf, r, pl.ds(_D + c * 16, 16)] = sig
                    if want_ep:
                        ep_v[buf, r, sl] = ep[c]
                return 0

            lax.fori_loop(0, _GE, row, 0)
            pltpu.async_copy(
                ms_v.at[buf], ms_h.at[pl.ds(base + j * _GE, _GE)], swbs[buf])
            if want_ep:
                pltpu.async_copy(
                    ep_v.at[buf], ep_h.at[pl.ds(base + j * _GE, _GE)],
                    swbs[buf])

        fetch(0, 0)

        def step(j, buf):
            @pl.when((j % 2) == buf)
            def _():
                @pl.when(j + 1 < nblk)
                def _():
                    fetch(j + 1, 1 - buf)
                wait_fetch(j, buf)
                compute(j, buf)

        def blk(j, _):
            step(j, 0)
            step(j, 1)
            return 0

        lax.fori_loop(0, nblk, blk, 0)
        for buf in range(2):
            pltpu.make_async_copy(
                ms_v.at[buf], ms_h.at[pl.ds(base, _GE)], swbs[buf]).wait()
            if want_ep:
                pltpu.make_async_copy(
                    ep_v.at[buf], ep_h.at[pl.ds(base, _GE)], swbs[buf]).wait()

    ms_type = jax.ShapeDtypeStruct((bb, 2 * _D), jnp.float32)
    common = [
        pltpu.VMEM((bpw,), jnp.int32),
        pltpu.VMEM((bpw,), jnp.int32),
        pltpu.VMEM((2, _GE, 2 * _D), jnp.float32),
        pltpu.VMEM((2, _GE, _D), jnp.float32),
        pltpu.VMEM((2, _GE, _D), jnp.float32),
        pltpu.VMEM((2, _GE, 2 * _D), jnp.float32),
    ]
    sems7 = [pltpu.SemaphoreType.DMA] * 8
    if want_ep:
        scratch = common + [pltpu.VMEM((2, _GE, _D), jnp.float32)] + sems7
        return pl.kernel(
            body,
            out_type=[ms_type, jax.ShapeDtypeStruct((bb, _D), jnp.float32)],
            mesh=mesh, compiler_params=_SC_PARAMS,
            scratch_types=scratch)(gu, gd, t, src, dst)
    scratch = common + sems7
    return pl.kernel(
        body, out_type=ms_type, mesh=mesh, compiler_params=_SC_PARAMS,
        scratch_types=scratch)(gu, gd, t, src, dst)


_SCH = 4000


def _scatter_small(ms, dst, n):
    bb = dst.shape[0]
    rpw = -(-(-(-n // _NW)) // 8) * 8
    acc_rows = rpw + 8
    nch = bb // _SCH
    assert bb % _SCH == 0 and _SCH % 16 == 0
    mesh = plsc.VectorSubcoreMesh(core_axis_name="c", subcore_axis_name="s")

    def body(ms_h, dst_h, ad_h, dst_v, cidx, ridx, rows_v, acc_v, sem):
        cid = lax.axis_index("c")
        sid = lax.axis_index("s")
        wid = sid * 2 + cid
        lo = wid * rpw
        hi = lo + rpw
        z16 = jnp.zeros((16,), jnp.float32)

        def zacc(i, _):
            acc_v[i // 16, pl.ds((i % 16) * 16, 16)] = z16
            return 0

        lax.fori_loop(0, acc_rows * 16, zacc, 0)
        trash16 = jnp.full((16,), acc_rows - 1, jnp.int32)
        zi16 = jnp.zeros((16,), jnp.int32)

        def chunk(ch, _):
            pltpu.sync_copy(dst_h.at[pl.ds(ch * _SCH, _SCH)], dst_v)

            def scan(v, cnt):
                d16 = dst_v[pl.ds(v * 16, 16)]
                m = (d16 >= lo) & (d16 < hi)
                eid = ch * _SCH + v * 16 + lax.iota(jnp.int32, 16)
                plsc.store_compressed(cidx.at[pl.ds(cnt, 16)], eid, mask=m)
                plsc.store_compressed(ridx.at[pl.ds(cnt, 16)], d16 - lo,
                                      mask=m)
                return cnt + jnp.sum(m.astype(jnp.int32))

            cnt = lax.fori_loop(0, _SCH // 16, scan, jnp.int32(0))
            for tp in range(_GB // 16):
                cidx[pl.ds(cnt + tp * 16, 16)] = zi16
                ridx[pl.ds(cnt + tp * 16, 16)] = trash16
            nb = (cnt + _GB - 1) // _GB

            def gs(gb, _):
                pltpu.async_copy(
                    ms_h.at[cidx.at[pl.ds(gb * _GB, _GB)]], rows_v,
                    sem).wait()

                def accrow(p, _):
                    p16 = jnp.broadcast_to(gb * _GB + p, (16,))
                    r = jnp.min(plsc.load_gather(ridx, [p16]))
                    for c in range(16):
                        sl = pl.ds(c * 16, 16)
                        acc_v[r, sl] = acc_v[r, sl] + rows_v[p, sl]
                    return 0

                lax.fori_loop(0, _GB, accrow, 0)
                return 0

            lax.fori_loop(0, nb, gs, 0)
            return 0

        lax.fori_loop(0, nch, chunk, 0)
        pltpu.sync_copy(acc_v.at[pl.ds(0, rpw)], ad_h.at[pl.ds(lo, rpw)])

    return pl.kernel(
        body,
        out_type=jax.ShapeDtypeStruct((_NW * rpw, 2 * _D), jnp.float32),
        mesh=mesh,
        compiler_params=_SC_PARAMS,
        scratch_types=[
            pltpu.VMEM((_SCH,), jnp.int32),
            pltpu.VMEM((_SCH + _GB,), jnp.int32),
            pltpu.VMEM((_SCH + _GB,), jnp.int32),
            pltpu.VMEM((_GB, 2 * _D), jnp.float32),
            pltpu.VMEM((acc_rows, 2 * _D), jnp.float32),
            pltpu.SemaphoreType.DMA,
        ],
    )(ms, dst)


def _scatter_phase(ms, dst, n):
    bb = dst.shape[0]
    ept = bb // 16
    nvr = ept // 16
    h = n // 2
    assert ept % 16 == 0 and n % 16 == 0 and h % 8 == 0
    zt = -(-(h // 16) // 8) * 8
    mesh = plsc.VectorSubcoreMesh(core_axis_name="c", subcore_axis_name="s")

    def body(ms_h, dst_h, ad_h, dst_v, cidx, gidx, sidx, rows_v, ard_v,
             zero_v, starts, cnts, sem, sem2):
        cid = lax.axis_index("c")
        sid = lax.axis_index("s")
        wid = sid * 2 + cid
        base = sid * ept
        pltpu.sync_copy(dst_h.at[pl.ds(base, ept)], dst_v)
        z16 = jnp.zeros((16,), jnp.float32)

        def zv(i, _):
            zero_v[i // 16, pl.ds((i % 16) * 16, 16)] = z16
            return 0

        lax.fori_loop(0, 16 * 16, zv, 0)

        def zero_blk(zi, _):
            off = cid * h + sid * zt + zi * 8

            @pl.when(off < (cid + 1) * h)
            def _():
                pltpu.sync_copy(zero_v.at[pl.ds(0, 8)],
                                ad_h.at[pl.ds(off, 8)])
            return 0

        lax.fori_loop(0, zt // 8, zero_blk, 0)
        pltpu.sync_copy(zero_v.at[pl.ds(0, 8)],
                        ad_h.at[pl.ds(n + wid * 8, 8)])

        lo = cid * h
        hi = lo + h
        cnt = jnp.int32(0)
        for b in range(16):
            starts[b] = cnt

            def scan(v, cnt):
                d16 = dst_v[pl.ds(v * 16, 16)]
                m = ((d16 & 15) == b) & (d16 >= lo) & (d16 < hi)
                lid = v * 16 + lax.iota(jnp.int32, 16)
                plsc.store_compressed(cidx.at[pl.ds(cnt, 16)], lid, mask=m)
                return cnt + jnp.sum(m.astype(jnp.int32))

            cnt = lax.fori_loop(0, nvr, scan, cnt)
            cnts[b] = cnt - starts[b]
            cnt = ((cnt + 15) // 16) * 16
        plsc.subcore_barrier()

        trash = jnp.int32(n) + wid * 8
        for q in range(16):
            bq = (sid + q) % 16
            st = starts[bq]
            cntb = cnts[bq]
            nb = (cntb + _GB - 1) // _GB

            def gs(gb, _):
                for p in range(_GB // 16):
                    pos = gb * _GB + p * 16
                    c16 = cidx[pl.ds(st + pos, 16)]
                    sel = (pos + lax.iota(jnp.int32, 16)) < cntb
                    c16 = jnp.where(sel, c16, 0)
                    gidx[pl.ds(p * 16, 16)] = c16 + base
                    li = plsc.load_gather(dst_v, [c16])
                    sidx[pl.ds(p * 16, 16)] = jnp.where(sel, li, trash)
                cpm = pltpu.async_copy(ms_h.at[gidx], rows_v, sem)
                cpa = pltpu.async_copy(ad_h.at[sidx], ard_v, sem2)
                cpm.wait()
                cpa.wait()

                iota16 = lax.iota(jnp.int32, 16)
                lane0 = iota16 == 0
                trash16 = jnp.broadcast_to(trash, (16,))

                def merge(p, _):
                    p16 = jnp.broadcast_to(p, (16,))
                    d16 = plsc.load_gather(sidx, [p16])
                    fp = p
                    for k in range(_GB // 16):
                        s16 = sidx[pl.ds(k * 16, 16)]
                        qidx = iota16 + (k * 16)
                        cand = jnp.where((s16 == d16) & (qidx < p), qidx,
                                         jnp.int32(_GB))
                        fp = jnp.minimum(fp, jnp.min(cand))
                    for c in range(16):
                        sl = pl.ds(c * 16, 16)
                        ard_v[fp, sl] = ard_v[fp, sl] + rows_v[p, sl]
                    keep = jnp.broadcast_to(fp == p, (16,))
                    plsc.store_scatter(sidx, [p16],
                                       jnp.where(keep, d16, trash16),
                                       mask=lane0)
                    return 0

                lax.fori_loop(0, _GB, merge, 0)
                pltpu.sync_copy(ard_v, ad_h.at[sidx])
                return 0

            lax.fori_loop(0, nb, gs, 0)
            plsc.subcore_barrier()

    return pl.kernel(
        body,
        out_type=jax.ShapeDtypeStruct((n + 256, 2 * _D), jnp.float32),
        mesh=mesh,
        compiler_params=_SC_PARAMS,
        scratch_types=[
            pltpu.VMEM((ept,), jnp.int32),
            pltpu.VMEM((ept + 320 + _GB,), jnp.int32),
            pltpu.VMEM((_GB,), jnp.int32),
            pltpu.VMEM((_GB,), jnp.int32),
            pltpu.VMEM((_GB, 2 * _D), jnp.float32),
            pltpu.VMEM((_GB, 2 * _D), jnp.float32),
            pltpu.VMEM((16, 2 * _D), jnp.float32),
            pltpu.SMEM((17,), jnp.int32),
            pltpu.SMEM((16,), jnp.int32),
            pltpu.SemaphoreType.DMA,
            pltpu.SemaphoreType.DMA,
        ],
    )(ms, dst)


def kernel(atom_feats, bond_attr, triplet_feats, h_periodic_complete,
           edge_index, angle_index, tuple_edge_index, params):
    pa, pb, pf = params['angle'], params['atom'], params['fuse']
    n = atom_feats.shape[0]
    e = bond_attr.shape[0]

    gu1, gd1, hd1 = _node_tables(bond_attr, pa)
    t1 = _mm(triplet_feats, pa['Weg'], pa['beg'])
    ms1, ep1 = _edge_phase(gu1, gd1, t1, angle_index[0], angle_index[1])
    triplet_upd = _edge_out(triplet_feats, ep1, pa['ln_e_g'], pa['ln_e_b'])
    ad1 = _scatter_phase(ms1, angle_index[1], e)
    bond_local, t2 = _node_upd_t2(hd1, ad1, bond_attr,
                                  pa['ln_n_g'], pa['ln_n_b'],
                                  pb['Weg'], pb['beg'])

    gua, gda, hda = _node_tables(atom_feats, pb)
    t3 = _mm(h_periodic_complete, pb['Weg'], pb['beg'])

    ms2, ep2 = _edge_phase(gua, gda, t2, edge_index[0], edge_index[1])
    bond_upd = _edge_out(bond_local, ep2, pb['ln_e_g'], pb['ln_e_b'])
    ad2 = _scatter_small(ms2, edge_index[1], n)[:n]

    ms3 = _edge_phase(gua, gda, t3, tuple_edge_index[0],
                      tuple_edge_index[1], want_ep=False)
    ad3 = _scatter_small(ms3, tuple_edge_index[1], n)[:n]

    final = _fusion(atom_feats, hda, ad2, ad3,
                    pb['ln_n_g'], pb['ln_n_b'], pf)
    return (final, bond_upd, triplet_upd)


# speedup vs baseline: 1.3707x; 1.3707x over previous
import functools

import jax
import jax.numpy as jnp
from jax import lax
from jax.experimental import pallas as pl
from jax.experimental.pallas import tpu as pltpu
from jax.experimental.pallas import tpu_sc as plsc

_D = 128
_NW = 32
_GE = 40
_GB = 64
_SC_PARAMS = pltpu.CompilerParams(needs_layout_passes=False)


def _mm_body(x_ref, w_ref, b_ref, o_ref):
    o_ref[...] = (
        jnp.dot(x_ref[...], w_ref[...], preferred_element_type=jnp.float32)
        + b_ref[...]
    )


def _mm(x, w, b, br=1000):
    r, d = x.shape
    k = w.shape[1]
    assert r % br == 0
    return pl.pallas_call(
        _mm_body,
        grid=(r // br,),
        in_specs=[
            pl.BlockSpec((br, d), lambda i: (i, 0)),
            pl.BlockSpec((d, k), lambda i: (0, 0)),
            pl.BlockSpec((1, k), lambda i: (0, 0)),
        ],
        out_specs=pl.BlockSpec((br, k), lambda i: (i, 0)),
        out_shape=jax.ShapeDtypeStruct((r, k), jnp.float32),
    )(x, w, b.reshape(1, k))


def _tables_body(x_ref, wgu_ref, bgu_ref, wgd_ref, bgd_ref, whd_ref, bhd_ref,
                 gu_ref, gd_ref, hd_ref):
    x = x_ref[...]
    gu_ref[...] = jnp.dot(x, wgu_ref[...], preferred_element_type=jnp.float32) + bgu_ref[...]
    gd_ref[...] = jnp.dot(x, wgd_ref[...], preferred_element_type=jnp.float32) + bgd_ref[...]
    hd_ref[...] = jnp.dot(x, whd_ref[...], preferred_element_type=jnp.float32) + bhd_ref[...]


def _node_tables(x, p, br=1000):
    r, d = x.shape
    assert r % br == 0
    wgu = jnp.concatenate([p['Wsg'], p['Wsu']], axis=1)
    bgu = jnp.concatenate([p['bsg'], p['bsu']]).reshape(1, 2 * d)
    return pl.pallas_call(
        _tables_body,
        grid=(r // br,),
        in_specs=[
            pl.BlockSpec((br, d), lambda i: (i, 0)),
            pl.BlockSpec((d, 2 * d), lambda i: (0, 0)),
            pl.BlockSpec((1, 2 * d), lambda i: (0, 0)),
            pl.BlockSpec((d, d), lambda i: (0, 0)),
            pl.BlockSpec((1, d), lambda i: (0, 0)),
            pl.BlockSpec((d, d), lambda i: (0, 0)),
            pl.BlockSpec((1, d), lambda i: (0, 0)),
        ],
        out_specs=[
            pl.BlockSpec((br, 2 * d), lambda i: (i, 0)),
            pl.BlockSpec((br, d), lambda i: (i, 0)),
            pl.BlockSpec((br, d), lambda i: (i, 0)),
        ],
        out_shape=[
            jax.ShapeDtypeStruct((r, 2 * d), jnp.float32),
            jax.ShapeDtypeStruct((r, d), jnp.float32),
            jax.ShapeDtypeStruct((r, d), jnp.float32),
        ],
    )(x, wgu, bgu, p['Wdg'], p['bdg'].reshape(1, d), p['Wdu'], p['bdu'].reshape(1, d))


def _ln(x, g=None, b=None, eps=1e-5):
    m = jnp.mean(x, axis=-1, keepdims=True)
    v = jnp.mean((x - m) ** 2, axis=-1, keepdims=True)
    y = (x - m) * lax.rsqrt(v + eps)
    if g is not None:
        y = y * g + b
    return y


def _silu(x):
    return x * jax.nn.sigmoid(x)


def _node_upd_t2_body(hd_ref, ad_ref, node_ref, g_ref, b_ref, w_ref, be_ref,
                      bl_ref, t2_ref):
    d = hd_ref.shape[-1]
    ad = ad_ref[...]
    h = hd_ref[...] + ad[:, :d] / (ad[:, d:] + 1e-6)
    bl = node_ref[...] + _silu(_ln(h, g_ref[...], b_ref[...]))
    bl_ref[...] = bl
    t2_ref[...] = jnp.dot(bl, w_ref[...], preferred_element_type=jnp.float32) + be_ref[...]


def _node_upd_t2(hd, ad, node, g, b, weg, beg, br=1000):
    r, d = hd.shape
    return pl.pallas_call(
        _node_upd_t2_body,
        grid=(r // br,),
        in_specs=[
            pl.BlockSpec((br, d), lambda i: (i, 0)),
            pl.BlockSpec((br, 2 * d), lambda i: (i, 0)),
            pl.BlockSpec((br, d), lambda i: (i, 0)),
            pl.BlockSpec((1, d), lambda i: (0, 0)),
            pl.BlockSpec((1, d), lambda i: (0, 0)),
            pl.BlockSpec((d, d), lambda i: (0, 0)),
            pl.BlockSpec((1, d), lambda i: (0, 0)),
        ],
        out_specs=[
            pl.BlockSpec((br, d), lambda i: (i, 0)),
            pl.BlockSpec((br, d), lambda i: (i, 0)),
        ],
        out_shape=[
            jax.ShapeDtypeStruct((r, d), jnp.float32),
            jax.ShapeDtypeStruct((r, d), jnp.float32),
        ],
    )(hd, ad, node, g.reshape(1, d), b.reshape(1, d), weg, beg.reshape(1, d))


def _fusion_body(atom_ref, hd_ref, ad2_ref, ad3_ref, g_ref, b_ref,
                 wf1_ref, bf1_ref, g1_ref, b1_ref, wf2_ref, bf2_ref, o_ref):
    d = hd_ref.shape[-1]
    atom = atom_ref[...]
    hd = hd_ref[...]
    g, b = g_ref[...], b_ref[...]

    ad2 = ad2_ref[...]
    h2 = hd + ad2[:, :d] / (ad2[:, d:] + 1e-6)
    g2l = _ln(atom + _silu(_ln(h2, g, b)))

    ad3 = ad3_ref[...]
    h3 = hd + ad3[:, :d] / (ad3[:, d:] + 1e-6)
    glob = _ln(atom + _silu(_ln(h3, g, b)))

    gate = jnp.concatenate([g2l, glob], axis=-1)
    h = jnp.dot(gate, wf1_ref[...], preferred_element_type=jnp.float32) + bf1_ref[...]
    h = jax.nn.relu(_ln(h, g1_ref[...], b1_ref[...]))
    z = jax.nn.sigmoid(
        jnp.dot(h, wf2_ref[...], preferred_element_type=jnp.float32) + bf2_ref[...])
    o_ref[...] = z * g2l + (1.0 - z) * glob


def _fusion(atom, hd, ad2, ad3, g, b, f, br=1000):
    r, d = atom.shape
    return pl.pallas_call(
        _fusion_body,
        grid=(r // br,),
        in_specs=[
            pl.BlockSpec((br, d), lambda i: (i, 0)),
            pl.BlockSpec((br, d), lambda i: (i, 0)),
            pl.BlockSpec((br, 2 * d), lambda i: (i, 0)),
            pl.BlockSpec((br, 2 * d), lambda i: (i, 0)),
            pl.BlockSpec((1, d), lambda i: (0, 0)),
            pl.BlockSpec((1, d), lambda i: (0, 0)),
            pl.BlockSpec((2 * d, d), lambda i: (0, 0)),
            pl.BlockSpec((1, d), lambda i: (0, 0)),
            pl.BlockSpec((1, d), lambda i: (0, 0)),
            pl.BlockSpec((1, d), lambda i: (0, 0)),
            pl.BlockSpec((d, d), lambda i: (0, 0)),
            pl.BlockSpec((1, d), lambda i: (0, 0)),
        ],
        out_specs=pl.BlockSpec((br, d), lambda i: (i, 0)),
        out_shape=jax.ShapeDtypeStruct((r, d), jnp.float32),
    )(atom, hd, ad2, ad3, g.reshape(1, d), b.reshape(1, d),
      f['Wf1'], f['bf1'].reshape(1, d), f['g1'].reshape(1, d),
      f['b1'].reshape(1, d), f['Wf2'], f['bf2'].reshape(1, d))


def _rsqrt16(x):
    i = plsc.bitcast(x, jnp.int32)
    i = jnp.int32(0x5F3759DF) - (i >> 1)
    y = plsc.bitcast(i, jnp.float32)
    for _ in range(4):
        y = y * (1.5 - 0.5 * x * y * y)
    return y


def _sig16(x):
    return 1.0 / (1.0 + jnp.exp(-x))


def _edge_out_body(ef_ref, ep_ref, g_ref, b_ref, o_ref):
    o_ref[...] = ef_ref[...] + _silu(_ln(ep_ref[...], g_ref[...], b_ref[...]))


def _edge_out(ef, ep, g, b, br=1000):
    r, d = ef.shape
    return pl.pallas_call(
        _edge_out_body,
        grid=(r // br,),
        in_specs=[
            pl.BlockSpec((br, d), lambda i: (i, 0)),
            pl.BlockSpec((br, d), lambda i: (i, 0)),
            pl.BlockSpec((1, d), lambda i: (0, 0)),
            pl.BlockSpec((1, d), lambda i: (0, 0)),
        ],
        out_specs=pl.BlockSpec((br, d), lambda i: (i, 0)),
        out_shape=jax.ShapeDtypeStruct((r, d), jnp.float32),
    )(ef, ep, g.reshape(1, d), b.reshape(1, d))


def _edge_phase(gu, gd, t, src, dst, want_ep=True):
    bb = t.shape[0]
    bpw = bb // _NW
    nblk = bpw // _GE
    assert bpw % _GE == 0 and (bpw % 8) == 0
    mesh = plsc.VectorSubcoreMesh(core_axis_name="c", subcore_axis_name="s")

    def body(*refs):
        if want_ep:
            (gu_h, gd_h, t_h, src_h, dst_h, ms_h, ep_h,
             src_v, dst_v, gu_v, gd_v, t_v, ms_v, ep_v,
             sgu0, sgu1, sgd0, sgd1, st0, st1, swb0, swb1) = refs
        else:
            (gu_h, gd_h, t_h, src_h, dst_h, ms_h,
             src_v, dst_v, gu_v, gd_v, t_v, ms_v,
             sgu0, sgu1, sgd0, sgd1, st0, st1, swb0, swb1) = refs
        sems = [(sgu0, sgd0, st0), (sgu1, sgd1, st1)]
        swbs = [swb0, swb1]
        wid = lax.axis_index("s") * 2 + lax.axis_index("c")
        base = wid * bpw
        pltpu.sync_copy(src_h.at[pl.ds(base, bpw)], src_v)
        pltpu.sync_copy(dst_h.at[pl.ds(base, bpw)], dst_v)

        def fetch(j, buf):
            s1, s2, s3 = sems[buf]
            pltpu.async_copy(
                gu_h.at[src_v.at[pl.ds(j * _GE, _GE)]], gu_v.at[buf], s1)
            pltpu.async_copy(
                gd_h.at[dst_v.at[pl.ds(j * _GE, _GE)]], gd_v.at[buf], s2)
            pltpu.async_copy(
                t_h.at[pl.ds(base + j * _GE, _GE)], t_v.at[buf], s3)

        def wait_fetch(j, buf):
            s1, s2, s3 = sems[buf]
            pltpu.make_async_copy(
                gu_h.at[src_v.at[pl.ds(j * _GE, _GE)]], gu_v.at[buf],
                s1).wait()
            pltpu.make_async_copy(
                gd_h.at[dst_v.at[pl.ds(j * _GE, _GE)]], gd_v.at[buf],
                s2).wait()
            pltpu.make_async_copy(
                t_h.at[pl.ds(base + j * _GE, _GE)], t_v.at[buf], s3).wait()

        def compute(j, buf):
            @pl.when(j >= 2)
            def _():
                pltpu.make_async_copy(
                    ms_v.at[buf], ms_h.at[pl.ds(base, _GE)], swbs[buf]).wait()
                if want_ep:
                    pltpu.make_async_copy(
                        ep_v.at[buf], ep_h.at[pl.ds(base, _GE)],
                        swbs[buf]).wait()

            def row(r, _):
                ep, uu = [], []
                for c in range(8):
                    sl = pl.ds(c * 16, 16)
                    ep.append(gu_v[buf, r, sl] + gd_v[buf, r, sl]
                              + t_v[buf, r, sl])
                    uu.append(gu_v[buf, r, pl.ds(_D + c * 16, 16)])
                for c in range(8):
                    sl = pl.ds(c * 16, 16)
                    sig = _sig16(ep[c])
                    ms_v[buf, r, sl] = sig * uu[c]
                    ms_v[buf, r, pl.ds(_D + c * 16, 16)] = sig
                    if want_ep:
                        ep_v[buf, r, sl] = ep[c]
                return 0

            lax.fori_loop(0, _GE, row, 0)
            pltpu.async_copy(
                ms_v.at[buf], ms_h.at[pl.ds(base + j * _GE, _GE)], swbs[buf])
            if want_ep:
                pltpu.async_copy(
                    ep_v.at[buf], ep_h.at[pl.ds(base + j * _GE, _GE)],
                    swbs[buf])

        fetch(0, 0)

        def step(j, buf):
            @pl.when((j % 2) == buf)
            def _():
                @pl.when(j + 1 < nblk)
                def _():
                    fetch(j + 1, 1 - buf)
                wait_fetch(j, buf)
                compute(j, buf)

        def blk(j, _):
            step(j, 0)
            step(j, 1)
            return 0

        lax.fori_loop(0, nblk, blk, 0)
        for buf in range(2):
            pltpu.make_async_copy(
                ms_v.at[buf], ms_h.at[pl.ds(base, _GE)], swbs[buf]).wait()
            if want_ep:
                pltpu.make_async_copy(
                    ep_v.at[buf], ep_h.at[pl.ds(base, _GE)], swbs[buf]).wait()

    ms_type = jax.ShapeDtypeStruct((bb, 2 * _D), jnp.float32)
    common = [
        pltpu.VMEM((bpw,), jnp.int32),
        pltpu.VMEM((bpw,), jnp.int32),
        pltpu.VMEM((2, _GE, 2 * _D), jnp.float32),
        pltpu.VMEM((2, _GE, _D), jnp.float32),
        pltpu.VMEM((2, _GE, _D), jnp.float32),
        pltpu.VMEM((2, _GE, 2 * _D), jnp.float32),
    ]
    sems7 = [pltpu.SemaphoreType.DMA] * 8
    if want_ep:
        scratch = common + [pltpu.VMEM((2, _GE, _D), jnp.float32)] + sems7
        return pl.kernel(
            body,
            out_type=[ms_type, jax.ShapeDtypeStruct((bb, _D), jnp.float32)],
            mesh=mesh, compiler_params=_SC_PARAMS,
            scratch_types=scratch)(gu, gd, t, src, dst)
    scratch = common + sems7
    return pl.kernel(
        body, out_type=ms_type, mesh=mesh, compiler_params=_SC_PARAMS,
        scratch_types=scratch)(gu, gd, t, src, dst)


def _scatter_phase(ms, dst, n):
    bb = dst.shape[0]
    ept = bb // 16
    nvr = ept // 16
    h = n // 2
    assert ept % 16 == 0 and n % 16 == 0 and h % 8 == 0
    zt = -(-(h // 16) // 8) * 8
    mesh = plsc.VectorSubcoreMesh(core_axis_name="c", subcore_axis_name="s")

    def body(ms_h, dst_h, ad_h, dst_v, cidx, gidx, sidx, rows_v, ard_v,
             zero_v, starts, cnts, sem, sem2):
        cid = lax.axis_index("c")
        sid = lax.axis_index("s")
        wid = sid * 2 + cid
        base = sid * ept
        pltpu.sync_copy(dst_h.at[pl.ds(base, ept)], dst_v)
        z16 = jnp.zeros((16,), jnp.float32)

        def zv(i, _):
            zero_v[i // 16, pl.ds((i % 16) * 16, 16)] = z16
            return 0

        lax.fori_loop(0, 16 * 16, zv, 0)

        def zero_blk(zi, _):
            off = cid * h + sid * zt + zi * 8

            @pl.when(off < (cid + 1) * h)
            def _():
                pltpu.sync_copy(zero_v.at[pl.ds(0, 8)],
                                ad_h.at[pl.ds(off, 8)])
            return 0

        lax.fori_loop(0, zt // 8, zero_blk, 0)
        pltpu.sync_copy(zero_v.at[pl.ds(0, 8)],
                        ad_h.at[pl.ds(n + wid * 8, 8)])

        lo = cid * h
        hi = lo + h
        cnt = jnp.int32(0)
        for b in range(16):
            starts[b] = cnt

            def scan(v, cnt):
                d16 = dst_v[pl.ds(v * 16, 16)]
                m = ((d16 & 15) == b) & (d16 >= lo) & (d16 < hi)
                lid = v * 16 + lax.iota(jnp.int32, 16)
                plsc.store_compressed(cidx.at[pl.ds(cnt, 16)], lid, mask=m)
                return cnt + jnp.sum(m.astype(jnp.int32))

            cnt = lax.fori_loop(0, nvr, scan, cnt)
            cnts[b] = cnt - starts[b]
            cnt = ((cnt + 15) // 16) * 16
        plsc.subcore_barrier()

        trash = jnp.int32(n) + wid * 8
        for q in range(16):
            bq = (sid + q) % 16
            st = starts[bq]
            cntb = cnts[bq]
            nb = (cntb + _GB - 1) // _GB

            def gs(gb, _):
                for p in range(_GB // 16):
                    pos = gb * _GB + p * 16
                    c16 = cidx[pl.ds(st + pos, 16)]
                    sel = (pos + lax.iota(jnp.int32, 16)) < cntb
                    c16 = jnp.where(sel, c16, 0)
                    gidx[pl.ds(p * 16, 16)] = c16 + base
                    li = plsc.load_gather(dst_v, [c16])
                    sidx[pl.ds(p * 16, 16)] = jnp.where(sel, li, trash)
                cpm = pltpu.async_copy(ms_h.at[gidx], rows_v, sem)
                cpa = pltpu.async_copy(ad_h.at[sidx], ard_v, sem2)
                cpm.wait()
                cpa.wait()

                iota16 = lax.iota(jnp.int32, 16)
                lane0 = iota16 == 0
                trash16 = jnp.broadcast_to(trash, (16,))

                def merge(p, _):
                    p16 = jnp.broadcast_to(p, (16,))
                    d16 = plsc.load_gather(sidx, [p16])
                    fp = p
                    for k in range(_GB // 16):
                        s16 = sidx[pl.ds(k * 16, 16)]
                        qidx = iota16 + (k * 16)
                        cand = jnp.where((s16 == d16) & (qidx < p), qidx,
                                         jnp.int32(_GB))
                        fp = jnp.minimum(fp, jnp.min(cand))
                    for c in range(16):
                        sl = pl.ds(c * 16, 16)
                        ard_v[fp, sl] = ard_v[fp, sl] + rows_v[p, sl]
                    keep = jnp.broadcast_to(fp == p, (16,))
                    plsc.store_scatter(sidx, [p16],
                                       jnp.where(keep, d16, trash16),
                                       mask=lane0)
                    return 0

                lax.fori_loop(0, _GB, merge, 0)
                pltpu.sync_copy(ard_v, ad_h.at[sidx])
                return 0

            lax.fori_loop(0, nb, gs, 0)
            plsc.subcore_barrier()

    return pl.kernel(
        body,
        out_type=jax.ShapeDtypeStruct((n + 256, 2 * _D), jnp.float32),
        mesh=mesh,
        compiler_params=_SC_PARAMS,
        scratch_types=[
            pltpu.VMEM((ept,), jnp.int32),
            pltpu.VMEM((ept + 320 + _GB,), jnp.int32),
            pltpu.VMEM((_GB,), jnp.int32),
            pltpu.VMEM((_GB,), jnp.int32),
            pltpu.VMEM((_GB, 2 * _D), jnp.float32),
            pltpu.VMEM((_GB, 2 * _D), jnp.float32),
            pltpu.VMEM((16, 2 * _D), jnp.float32),
            pltpu.SMEM((17,), jnp.int32),
            pltpu.SMEM((16,), jnp.int32),
            pltpu.SemaphoreType.DMA,
            pltpu.SemaphoreType.DMA,
        ],
    )(ms, dst)


def kernel(atom_feats, bond_attr, triplet_feats, h_periodic_complete,
           edge_index, angle_index, tuple_edge_index, params):
    pa, pb, pf = params['angle'], params['atom'], params['fuse']
    n = atom_feats.shape[0]
    e = bond_attr.shape[0]

    gu1, gd1, hd1 = _node_tables(bond_attr, pa)
    t1 = _mm(triplet_feats, pa['Weg'], pa['beg'])
    ms1, ep1 = _edge_phase(gu1, gd1, t1, angle_index[0], angle_index[1])
    triplet_upd = _edge_out(triplet_feats, ep1, pa['ln_e_g'], pa['ln_e_b'])
    ad1 = _scatter_phase(ms1, angle_index[1], e)
    bond_local, t2 = _node_upd_t2(hd1, ad1, bond_attr,
                                  pa['ln_n_g'], pa['ln_n_b'],
                                  pb['Weg'], pb['beg'])

    gua, gda, hda = _node_tables(atom_feats, pb)
    t3 = _mm(h_periodic_complete, pb['Weg'], pb['beg'])

    ms2, ep2 = _edge_phase(gua, gda, t2, edge_index[0], edge_index[1])
    bond_upd = _edge_out(bond_local, ep2, pb['ln_e_g'], pb['ln_e_b'])
    ad2 = _scatter_phase(ms2, edge_index[1], n)

    ms3 = _edge_phase(gua, gda, t3, tuple_edge_index[0],
                      tuple_edge_index[1], want_ep=False)
    ad3 = _scatter_phase(ms3, tuple_edge_index[1], n)

    final = _fusion(atom_feats, hda, ad2, ad3,
                    pb['ln_n_g'], pb['ln_n_b'], pf)
    return (final, bond_upd, triplet_upd)


# speedup vs baseline: 1.3823x; 1.0085x over previous
import functools

import jax
import jax.numpy as jnp
from jax import lax
from jax.experimental import pallas as pl
from jax.experimental.pallas import tpu as pltpu
from jax.experimental.pallas import tpu_sc as plsc

_D = 128
_NW = 32
_GE = 40
_GB = 64
_SC_PARAMS = pltpu.CompilerParams(needs_layout_passes=False)


def _mm_body(x_ref, w_ref, b_ref, o_ref):
    o_ref[...] = (
        jnp.dot(x_ref[...], w_ref[...], preferred_element_type=jnp.float32)
        + b_ref[...]
    )


def _mm(x, w, b, br=1000):
    r, d = x.shape
    k = w.shape[1]
    assert r % br == 0
    return pl.pallas_call(
        _mm_body,
        grid=(r // br,),
        in_specs=[
            pl.BlockSpec((br, d), lambda i: (i, 0)),
            pl.BlockSpec((d, k), lambda i: (0, 0)),
            pl.BlockSpec((1, k), lambda i: (0, 0)),
        ],
        out_specs=pl.BlockSpec((br, k), lambda i: (i, 0)),
        out_shape=jax.ShapeDtypeStruct((r, k), jnp.float32),
    )(x, w, b.reshape(1, k))


def _tables_body(x_ref, wgu_ref, bgu_ref, wgd_ref, bgd_ref, whd_ref, bhd_ref,
                 gu_ref, gd_ref, hd_ref):
    x = x_ref[...]
    gu_ref[...] = jnp.dot(x, wgu_ref[...], preferred_element_type=jnp.float32) + bgu_ref[...]
    gd_ref[...] = jnp.dot(x, wgd_ref[...], preferred_element_type=jnp.float32) + bgd_ref[...]
    hd_ref[...] = jnp.dot(x, whd_ref[...], preferred_element_type=jnp.float32) + bhd_ref[...]


def _node_tables(x, p, br=1000):
    r, d = x.shape
    assert r % br == 0
    wgu = jnp.concatenate([p['Wsg'], p['Wsu']], axis=1)
    bgu = jnp.concatenate([p['bsg'], p['bsu']]).reshape(1, 2 * d)
    return pl.pallas_call(
        _tables_body,
        grid=(r // br,),
        in_specs=[
            pl.BlockSpec((br, d), lambda i: (i, 0)),
            pl.BlockSpec((d, 2 * d), lambda i: (0, 0)),
            pl.BlockSpec((1, 2 * d), lambda i: (0, 0)),
            pl.BlockSpec((d, d), lambda i: (0, 0)),
            pl.BlockSpec((1, d), lambda i: (0, 0)),
            pl.BlockSpec((d, d), lambda i: (0, 0)),
            pl.BlockSpec((1, d), lambda i: (0, 0)),
        ],
        out_specs=[
            pl.BlockSpec((br, 2 * d), lambda i: (i, 0)),
            pl.BlockSpec((br, d), lambda i: (i, 0)),
            pl.BlockSpec((br, d), lambda i: (i, 0)),
        ],
        out_shape=[
            jax.ShapeDtypeStruct((r, 2 * d), jnp.float32),
            jax.ShapeDtypeStruct((r, d), jnp.float32),
            jax.ShapeDtypeStruct((r, d), jnp.float32),
        ],
    )(x, wgu, bgu, p['Wdg'], p['bdg'].reshape(1, d), p['Wdu'], p['bdu'].reshape(1, d))


def _ln(x, g=None, b=None, eps=1e-5):
    m = jnp.mean(x, axis=-1, keepdims=True)
    v = jnp.mean((x - m) ** 2, axis=-1, keepdims=True)
    y = (x - m) * lax.rsqrt(v + eps)
    if g is not None:
        y = y * g + b
    return y


def _silu(x):
    return x * jax.nn.sigmoid(x)


def _node_upd_t2_body(hd_ref, ad_ref, node_ref, g_ref, b_ref, w_ref, be_ref,
                      bl_ref, t2_ref):
    d = hd_ref.shape[-1]
    ad = ad_ref[...]
    h = hd_ref[...] + ad[:, :d] / (ad[:, d:] + 1e-6)
    bl = node_ref[...] + _silu(_ln(h, g_ref[...], b_ref[...]))
    bl_ref[...] = bl
    t2_ref[...] = jnp.dot(bl, w_ref[...], preferred_element_type=jnp.float32) + be_ref[...]


def _node_upd_t2(hd, ad, node, g, b, weg, beg, br=1000):
    r, d = hd.shape
    return pl.pallas_call(
        _node_upd_t2_body,
        grid=(r // br,),
        in_specs=[
            pl.BlockSpec((br, d), lambda i: (i, 0)),
            pl.BlockSpec((br, 2 * d), lambda i: (i, 0)),
            pl.BlockSpec((br, d), lambda i: (i, 0)),
            pl.BlockSpec((1, d), lambda i: (0, 0)),
            pl.BlockSpec((1, d), lambda i: (0, 0)),
            pl.BlockSpec((d, d), lambda i: (0, 0)),
            pl.BlockSpec((1, d), lambda i: (0, 0)),
        ],
        out_specs=[
            pl.BlockSpec((br, d), lambda i: (i, 0)),
            pl.BlockSpec((br, d), lambda i: (i, 0)),
        ],
        out_shape=[
            jax.ShapeDtypeStruct((r, d), jnp.float32),
            jax.ShapeDtypeStruct((r, d), jnp.float32),
        ],
    )(hd, ad, node, g.reshape(1, d), b.reshape(1, d), weg, beg.reshape(1, d))


def _fusion_body(atom_ref, hd_ref, ad2_ref, ad3_ref, g_ref, b_ref,
                 wf1_ref, bf1_ref, g1_ref, b1_ref, wf2_ref, bf2_ref, o_ref):
    d = hd_ref.shape[-1]
    atom = atom_ref[...]
    hd = hd_ref[...]
    g, b = g_ref[...], b_ref[...]

    ad2 = ad2_ref[...]
    h2 = hd + ad2[:, :d] / (ad2[:, d:] + 1e-6)
    g2l = _ln(atom + _silu(_ln(h2, g, b)))

    ad3 = ad3_ref[...]
    h3 = hd + ad3[:, :d] / (ad3[:, d:] + 1e-6)
    glob = _ln(atom + _silu(_ln(h3, g, b)))

    gate = jnp.concatenate([g2l, glob], axis=-1)
    h = jnp.dot(gate, wf1_ref[...], preferred_element_type=jnp.float32) + bf1_ref[...]
    h = jax.nn.relu(_ln(h, g1_ref[...], b1_ref[...]))
    z = jax.nn.sigmoid(
        jnp.dot(h, wf2_ref[...], preferred_element_type=jnp.float32) + bf2_ref[...])
    o_ref[...] = z * g2l + (1.0 - z) * glob


def _fusion(atom, hd, ad2, ad3, g, b, f, br=1000):
    r, d = atom.shape
    return pl.pallas_call(
        _fusion_body,
        grid=(r // br,),
        in_specs=[
            pl.BlockSpec((br, d), lambda i: (i, 0)),
            pl.BlockSpec((br, d), lambda i: (i, 0)),
            pl.BlockSpec((br, 2 * d), lambda i: (i, 0)),
            pl.BlockSpec((br, 2 * d), lambda i: (i, 0)),
            pl.BlockSpec((1, d), lambda i: (0, 0)),
            pl.BlockSpec((1, d), lambda i: (0, 0)),
            pl.BlockSpec((2 * d, d), lambda i: (0, 0)),
            pl.BlockSpec((1, d), lambda i: (0, 0)),
            pl.BlockSpec((1, d), lambda i: (0, 0)),
            pl.BlockSpec((1, d), lambda i: (0, 0)),
            pl.BlockSpec((d, d), lambda i: (0, 0)),
            pl.BlockSpec((1, d), lambda i: (0, 0)),
        ],
        out_specs=pl.BlockSpec((br, d), lambda i: (i, 0)),
        out_shape=jax.ShapeDtypeStruct((r, d), jnp.float32),
    )(atom, hd, ad2, ad3, g.reshape(1, d), b.reshape(1, d),
      f['Wf1'], f['bf1'].reshape(1, d), f['g1'].reshape(1, d),
      f['b1'].reshape(1, d), f['Wf2'], f['bf2'].reshape(1, d))


def _rsqrt16(x):
    i = plsc.bitcast(x, jnp.int32)
    i = jnp.int32(0x5F3759DF) - (i >> 1)
    y = plsc.bitcast(i, jnp.float32)
    for _ in range(4):
        y = y * (1.5 - 0.5 * x * y * y)
    return y


def _sig16(x):
    return 1.0 / (1.0 + jnp.exp(-x))


def _edge_out_body(ef_ref, ep_ref, g_ref, b_ref, o_ref):
    o_ref[...] = ef_ref[...] + _silu(_ln(ep_ref[...], g_ref[...], b_ref[...]))


def _edge_out(ef, ep, g, b, br=1000):
    r, d = ef.shape
    return pl.pallas_call(
        _edge_out_body,
        grid=(r // br,),
        in_specs=[
            pl.BlockSpec((br, d), lambda i: (i, 0)),
            pl.BlockSpec((br, d), lambda i: (i, 0)),
            pl.BlockSpec((1, d), lambda i: (0, 0)),
            pl.BlockSpec((1, d), lambda i: (0, 0)),
        ],
        out_specs=pl.BlockSpec((br, d), lambda i: (i, 0)),
        out_shape=jax.ShapeDtypeStruct((r, d), jnp.float32),
    )(ef, ep, g.reshape(1, d), b.reshape(1, d))


def _edge_phase(gu, gd, t, src, dst, want_ep=True):
    bb = t.shape[0]
    bpw = bb // _NW
    nblk = bpw // _GE
    assert bpw % _GE == 0 and (bpw % 8) == 0
    mesh = plsc.VectorSubcoreMesh(core_axis_name="c", subcore_axis_name="s")

    def body(*refs):
        if want_ep:
            (gu_h, gd_h, t_h, src_h, dst_h, ms_h, ep_h,
             src_v, dst_v, gu_v, gd_v, t_v, ms_v, ep_v,
             sgu0, sgu1, sgd0, sgd1, st0, st1, swb0, swb1) = refs
        else:
            (gu_h, gd_h, t_h, src_h, dst_h, ms_h,
             src_v, dst_v, gu_v, gd_v, t_v, ms_v,
             sgu0, sgu1, sgd0, sgd1, st0, st1, swb0, swb1) = refs
        sems = [(sgu0, sgd0, st0), (sgu1, sgd1, st1)]
        swbs = [swb0, swb1]
        wid = lax.axis_index("s") * 2 + lax.axis_index("c")
        base = wid * bpw
        pltpu.sync_copy(src_h.at[pl.ds(base, bpw)], src_v)
        pltpu.sync_copy(dst_h.at[pl.ds(base, bpw)], dst_v)

        def fetch(j, buf):
            s1, s2, s3 = sems[buf]
            pltpu.async_copy(
                gu_h.at[src_v.at[pl.ds(j * _GE, _GE)]], gu_v.at[buf], s1)
            pltpu.async_copy(
                gd_h.at[dst_v.at[pl.ds(j * _GE, _GE)]], gd_v.at[buf], s2)
            pltpu.async_copy(
                t_h.at[pl.ds(base + j * _GE, _GE)], t_v.at[buf], s3)

        def wait_fetch(j, buf):
            s1, s2, s3 = sems[buf]
            pltpu.make_async_copy(
                gu_h.at[src_v.at[pl.ds(j * _GE, _GE)]], gu_v.at[buf],
                s1).wait()
            pltpu.make_async_copy(
                gd_h.at[dst_v.at[pl.ds(j * _GE, _GE)]], gd_v.at[buf],
                s2).wait()
            pltpu.make_async_copy(
                t_h.at[pl.ds(base + j * _GE, _GE)], t_v.at[buf], s3).wait()

        def compute(j, buf):
            @pl.when(j >= 2)
            def _():
                pltpu.make_async_copy(
                    ms_v.at[buf], ms_h.at[pl.ds(base, _GE)], swbs[buf]).wait()
                if want_ep:
                    pltpu.make_async_copy(
                        ep_v.at[buf], ep_h.at[pl.ds(base, _GE)],
                        swbs[buf]).wait()

            def row(r, _):
                ep, uu = [], []
                for c in range(8):
                    sl = pl.ds(c * 16, 16)
                    ep.append(gu_v[buf, r, sl] + gd_v[buf, r, sl]
                              + t_v[buf, r, sl])
                    uu.append(gu_v[buf, r, pl.ds(_D + c * 16, 16)])
                for c in range(8):
                    sl = pl.ds(c * 16, 16)
                    sig = _sig16(ep[c])
                    ms_v[buf, r, sl] = sig * uu[c]
                    ms_v[buf, r, pl.ds(_D + c * 16, 16)] = sig
                    if want_ep:
                        ep_v[buf, r, sl] = ep[c]
                return 0

            lax.fori_loop(0, _GE, row, 0)
            pltpu.async_copy(
                ms_v.at[buf], ms_h.at[pl.ds(base + j * _GE, _GE)], swbs[buf])
            if want_ep:
                pltpu.async_copy(
                    ep_v.at[buf], ep_h.at[pl.ds(base + j * _GE, _GE)],
                    swbs[buf])

        fetch(0, 0)

        def step(j, buf):
            @pl.when((j % 2) == buf)
            def _():
                @pl.when(j + 1 < nblk)
                def _():
                    fetch(j + 1, 1 - buf)
                wait_fetch(j, buf)
                compute(j, buf)

        def blk(j, _):
            step(j, 0)
            step(j, 1)
            return 0

        lax.fori_loop(0, nblk, blk, 0)
        for buf in range(2):
            pltpu.make_async_copy(
                ms_v.at[buf], ms_h.at[pl.ds(base, _GE)], swbs[buf]).wait()
            if want_ep:
                pltpu.make_async_copy(
                    ep_v.at[buf], ep_h.at[pl.ds(base, _GE)], swbs[buf]).wait()

    ms_type = jax.ShapeDtypeStruct((bb, 2 * _D), jnp.float32)
    common = [
        pltpu.VMEM((bpw,), jnp.int32),
        pltpu.VMEM((bpw,), jnp.int32),
        pltpu.VMEM((2, _GE, 2 * _D), jnp.float32),
        pltpu.VMEM((2, _GE, _D), jnp.float32),
        pltpu.VMEM((2, _GE, _D), jnp.float32),
        pltpu.VMEM((2, _GE, 2 * _D), jnp.float32),
    ]
    sems7 = [pltpu.SemaphoreType.DMA] * 8
    if want_ep:
        scratch = common + [pltpu.VMEM((2, _GE, _D), jnp.float32)] + sems7
        return pl.kernel(
            body,
            out_type=[ms_type, jax.ShapeDtypeStruct((bb, _D), jnp.float32)],
            mesh=mesh, compiler_params=_SC_PARAMS,
            scratch_types=scratch)(gu, gd, t, src, dst)
    scratch = common + sems7
    return pl.kernel(
        body, out_type=ms_type, mesh=mesh, compiler_params=_SC_PARAMS,
        scratch_types=scratch)(gu, gd, t, src, dst)


def _scatter_phase(ms, dst, n):
    bb = dst.shape[0]
    ept = bb // 16
    nvr = ept // 16
    h = n // 2
    assert ept % 16 == 0 and n % 16 == 0 and h % 8 == 0
    zt = -(-(h // 16) // 8) * 8
    mesh = plsc.VectorSubcoreMesh(core_axis_name="c", subcore_axis_name="s")

    def body(ms_h, dst_h, ad_h, dst_v, cidx, gidx, sidx, rows_v, ard_v,
             zero_v, starts, cnts, sem, sem2):
        cid = lax.axis_index("c")
        sid = lax.axis_index("s")
        wid = sid * 2 + cid
        base = sid * ept
        pltpu.sync_copy(dst_h.at[pl.ds(base, ept)], dst_v)
        z16 = jnp.zeros((16,), jnp.float32)

        def zv(i, _):
            zero_v[i // 16, pl.ds((i % 16) * 16, 16)] = z16
            return 0

        lax.fori_loop(0, 16 * 16, zv, 0)

        def zero_blk(zi, _):
            off = cid * h + sid * zt + zi * 8

            @pl.when(off < (cid + 1) * h)
            def _():
                pltpu.sync_copy(zero_v.at[pl.ds(0, 8)],
                                ad_h.at[pl.ds(off, 8)])
            return 0

        lax.fori_loop(0, zt // 8, zero_blk, 0)
        pltpu.sync_copy(zero_v.at[pl.ds(0, 8)],
                        ad_h.at[pl.ds(n + wid * 8, 8)])

        lo = cid * h
        hi = lo + h
        cnt = jnp.int32(0)
        for b in range(16):
            starts[b] = cnt

            def scan(v, cnt):
                d16 = dst_v[pl.ds(v * 16, 16)]
                m = ((d16 & 15) == b) & (d16 >= lo) & (d16 < hi)
                lid = v * 16 + lax.iota(jnp.int32, 16)
                plsc.store_compressed(cidx.at[pl.ds(cnt, 16)], lid, mask=m)
                return cnt + jnp.sum(m.astype(jnp.int32))

            cnt = lax.fori_loop(0, nvr, scan, cnt)
            cnts[b] = cnt - starts[b]
            cnt = ((cnt + 15) // 16) * 16
        plsc.subcore_barrier()

        trash = jnp.int32(n) + wid * 8
        for q in range(16):
            bq = (sid + q) % 16
            st = starts[bq]
            cntb = cnts[bq]
            nb = (cntb + _GB - 1) // _GB

            def gs(gb, _):
                for p in range(_GB // 16):
                    pos = gb * _GB + p * 16
                    c16 = cidx[pl.ds(st + pos, 16)]
                    sel = (pos + lax.iota(jnp.int32, 16)) < cntb
                    c16 = jnp.where(sel, c16, 0)
                    gidx[pl.ds(p * 16, 16)] = c16 + base
                    li = plsc.load_gather(dst_v, [c16])
                    sidx[pl.ds(p * 16, 16)] = jnp.where(sel, li, trash)
                cpm = pltpu.async_copy(ms_h.at[gidx], rows_v, sem)
                cpa = pltpu.async_copy(ad_h.at[sidx], ard_v, sem2)
                cpm.wait()
                cpa.wait()

                iota16 = lax.iota(jnp.int32, 16)
                lane0 = iota16 == 0
                trash16 = jnp.broadcast_to(trash, (16,))

                def merge(p, _):
                    p16 = jnp.broadcast_to(p, (16,))
                    d16 = plsc.load_gather(sidx, [p16])
                    cmin = jnp.full((16,), _GB, jnp.int32)
                    for k in range(_GB // 16):
                        s16 = sidx[pl.ds(k * 16, 16)]
                        qidx = iota16 + (k * 16)
                        cand = jnp.where((s16 == d16) & (qidx < p), qidx,
                                         jnp.int32(_GB))
                        cmin = jnp.minimum(cmin, cand)
                    fp = jnp.minimum(p, jnp.min(cmin))
                    for c in range(16):
                        sl = pl.ds(c * 16, 16)
                        ard_v[fp, sl] = ard_v[fp, sl] + rows_v[p, sl]
                    keep = jnp.broadcast_to(fp == p, (16,))
                    plsc.store_scatter(sidx, [p16],
                                       jnp.where(keep, d16, trash16),
                                       mask=lane0)
                    return 0

                lax.fori_loop(0, _GB, merge, 0)
                pltpu.sync_copy(ard_v, ad_h.at[sidx])
                return 0

            lax.fori_loop(0, nb, gs, 0)
            plsc.subcore_barrier()

    return pl.kernel(
        body,
        out_type=jax.ShapeDtypeStruct((n + 256, 2 * _D), jnp.float32),
        mesh=mesh,
        compiler_params=_SC_PARAMS,
        scratch_types=[
            pltpu.VMEM((ept,), jnp.int32),
            pltpu.VMEM((ept + 320 + _GB,), jnp.int32),
            pltpu.VMEM((_GB,), jnp.int32),
            pltpu.VMEM((_GB,), jnp.int32),
            pltpu.VMEM((_GB, 2 * _D), jnp.float32),
            pltpu.VMEM((_GB, 2 * _D), jnp.float32),
            pltpu.VMEM((16, 2 * _D), jnp.float32),
            pltpu.SMEM((17,), jnp.int32),
            pltpu.SMEM((16,), jnp.int32),
            pltpu.SemaphoreType.DMA,
            pltpu.SemaphoreType.DMA,
        ],
    )(ms, dst)


def kernel(atom_feats, bond_attr, triplet_feats, h_periodic_complete,
           edge_index, angle_index, tuple_edge_index, params):
    pa, pb, pf = params['angle'], params['atom'], params['fuse']
    n = atom_feats.shape[0]
    e = bond_attr.shape[0]

    gu1, gd1, hd1 = _node_tables(bond_attr, pa)
    t1 = _mm(triplet_feats, pa['Weg'], pa['beg'])
    ms1, ep1 = _edge_phase(gu1, gd1, t1, angle_index[0], angle_index[1])
    triplet_upd = _edge_out(triplet_feats, ep1, pa['ln_e_g'], pa['ln_e_b'])
    ad1 = _scatter_phase(ms1, angle_index[1], e)
    bond_local, t2 = _node_upd_t2(hd1, ad1, bond_attr,
                                  pa['ln_n_g'], pa['ln_n_b'],
                                  pb['Weg'], pb['beg'])

    gua, gda, hda = _node_tables(atom_feats, pb)
    t3 = _mm(h_periodic_complete, pb['Weg'], pb['beg'])

    ms2, ep2 = _edge_phase(gua, gda, t2, edge_index[0], edge_index[1])
    bond_upd = _edge_out(bond_local, ep2, pb['ln_e_g'], pb['ln_e_b'])
    ad2 = _scatter_phase(ms2, edge_index[1], n)

    ms3 = _edge_phase(gua, gda, t3, tuple_edge_index[0],
                      tuple_edge_index[1], want_ep=False)
    ad3 = _scatter_phase(ms3, tuple_edge_index[1], n)

    final = _fusion(atom_feats, hda, ad2, ad3,
                    pb['ln_n_g'], pb['ln_n_b'], pf)
    return (final, bond_upd, triplet_upd)
